# baseline (speedup 1.0000x reference)
import functools

import jax
import jax.numpy as jnp
from jax import lax
from jax.experimental import pallas as pl
from jax.experimental.pallas import tpu as pltpu
from jax.experimental.pallas import tpu_sc as plsc

VOCAB = 100000
EMBED = 128
NPOS = 2048
B = 4
S = 2048

NC = 2
NS = 16
NW = NC * NS
NROWS = B * S
ROWS_PER_W = NROWS // NW
WPB = S // ROWS_PER_W
NWIN = 4
NCHUNK = 8
CH = ROWS_PER_W // NCHUNK
LANES = 16

_mesh = plsc.VectorSubcoreMesh(
    core_axis_name="c", subcore_axis_name="s", num_cores=NC, num_subcores=NS
)


@functools.partial(
    pl.kernel,
    out_type=jax.ShapeDtypeStruct((B, S, EMBED), jnp.float32),
    mesh=_mesh,
    scratch_types=[
        pltpu.VMEM((ROWS_PER_W,), jnp.int32),
        pltpu.VMEM((NCHUNK, CH), jnp.int32),
        pltpu.VMEM((ROWS_PER_W, EMBED), jnp.float32),
        pltpu.VMEM((ROWS_PER_W, EMBED), jnp.float32),
        pltpu.VMEM_SHARED((NS * ROWS_PER_W, EMBED), jnp.float32),
        pltpu.VMEM_SHARED((NWIN * ROWS_PER_W, EMBED), jnp.float32),
        pltpu.SemaphoreType.DMA,
        pltpu.SemaphoreType.DMA,
        pltpu.SemaphoreType.DMA,
        pltpu.SemaphoreType.DMA,
        pltpu.SemaphoreType.DMA,
        pltpu.SemaphoreType.DMA,
        pltpu.SemaphoreType.DMA,
        pltpu.SemaphoreType.DMA,
        pltpu.SemaphoreType.DMA,
        pltpu.SemaphoreType.DMA,
        pltpu.SemaphoreType.DMA,
        pltpu.SemaphoreType.DMA,
    ],
)
def _embed_kernel(x_hbm, tok_hbm, pos_hbm, out_hbm, idx_v, ids_v, tok_v,
                  pos_v, acc_sh, pos_sh, sem_in, sem_stage, sem_pos, sem0,
                  sem1, sem2, sem3, sem4, sem5, sem6, sem7, sem_out):
    sems = [sem0, sem1, sem2, sem3, sem4, sem5, sem6, sem7]
    cid = lax.axis_index("c")
    sid = lax.axis_index("s")
    wid = sid * NC + cid
    b = wid // WPB
    s0 = lax.rem(wid, WPB) * ROWS_PER_W
    spbase = sid * ROWS_PER_W
    lw = lax.rem(sid, NWIN)

    idx_cp = pltpu.async_copy(x_hbm.at[b, pl.ds(s0, ROWS_PER_W)], idx_v, sem_in)

    @pl.when(sid < NWIN)
    def _stage_pos():
        pltpu.async_copy(
            pos_hbm.at[pl.ds(s0, ROWS_PER_W)],
            pos_sh.at[pl.ds(sid * ROWS_PER_W, ROWS_PER_W)],
            sem_stage,
        ).wait()

    lane = lax.iota(jnp.int32, 16)
    for j in range(NCHUNK):
        for k in range(CH // LANES):
            ids_v[j, pl.ds(k * LANES, LANES)] = lane + (
                spbase + j * CH + k * LANES
            )

    plsc.subcore_barrier()
    pos_cp = pltpu.async_copy(
        pos_sh.at[pl.ds(lw * ROWS_PER_W, ROWS_PER_W)], pos_v, sem_pos
    )

    idx_cp.wait()
    g_cps = [
        pltpu.async_copy(
            tok_hbm.at[idx_v.at[pl.ds(c * CH, CH)]],
            tok_v.at[pl.ds(c * CH, CH)],
            sems[c],
        )
        for c in range(NCHUNK)
    ]
    pos_cp.wait()

    init_cps = [None] * NCHUNK
    sa_cps = [None] * NCHUNK
    out_cp = None
    for t in range(NCHUNK + 2):
        if t < NCHUNK:
            g_cps[t].wait()
            init_cps[t] = pltpu.async_copy(
                tok_v.at[pl.ds(t * CH, CH)],
                acc_sh.at[pl.ds(spbase + t * CH, CH)],
                sems[t],
            )
        if 0 <= t - 1 < NCHUNK:
            c = t - 1
            init_cps[c].wait()
            sa_cps[c] = pltpu.async_copy(
                pos_v.at[pl.ds(c * CH, CH)],
                acc_sh.at[ids_v.at[c]],
                sems[c],
                add=True,
            )
        if 0 <= t - 2 < NCHUNK:
            c = t - 2
            sa_cps[c].wait()
            out_cp = pltpu.async_copy(
                acc_sh.at[pl.ds(spbase + c * CH, CH)],
                out_hbm.at[b, pl.ds(s0 + c * CH, CH)],
                sem_out,
            )
    for _ in range(NCHUNK):
        out_cp.wait()


def kernel(x, tok_table, pos_table):
    return _embed_kernel(x, tok_table, pos_table)


# speedup vs baseline: 1.0165x; 1.0165x over previous
import functools

import jax
import jax.numpy as jnp
from jax import lax
from jax.experimental import pallas as pl
from jax.experimental.pallas import tpu as pltpu
from jax.experimental.pallas import tpu_sc as plsc

VOCAB = 100000
EMBED = 128
NPOS = 2048
B = 4
S = 2048

NC = 2
NS = 16
NW = NC * NS
NROWS = B * S
ROWS_PER_W = NROWS // NW
WPB = S // ROWS_PER_W
NCHUNK = 8
CH = ROWS_PER_W // NCHUNK
LANES = 16

_mesh = plsc.VectorSubcoreMesh(
    core_axis_name="c", subcore_axis_name="s", num_cores=NC, num_subcores=NS
)


@functools.partial(
    pl.kernel,
    out_type=jax.ShapeDtypeStruct((B, S, EMBED), jnp.float32),
    mesh=_mesh,
    scratch_types=[
        pltpu.VMEM((ROWS_PER_W,), jnp.int32),
        pltpu.VMEM((NCHUNK, CH), jnp.int32),
        pltpu.VMEM((ROWS_PER_W, EMBED), jnp.float32),
        pltpu.VMEM_SHARED((NS * ROWS_PER_W, EMBED), jnp.float32),
        pltpu.SemaphoreType.DMA,
        pltpu.SemaphoreType.DMA,
        pltpu.SemaphoreType.DMA,
        pltpu.SemaphoreType.DMA,
        pltpu.SemaphoreType.DMA,
        pltpu.SemaphoreType.DMA,
        pltpu.SemaphoreType.DMA,
        pltpu.SemaphoreType.DMA,
        pltpu.SemaphoreType.DMA,
        pltpu.SemaphoreType.DMA,
    ],
)
def _embed_kernel(x_hbm, tok_hbm, pos_hbm, out_hbm, idx_v, ids_v, tok_v,
                  acc_sh, sem_in, sem0, sem1, sem2, sem3, sem4, sem5, sem6,
                  sem7, sem_out):
    sems = [sem0, sem1, sem2, sem3, sem4, sem5, sem6, sem7]
    cid = lax.axis_index("c")
    sid = lax.axis_index("s")
    wid = sid * NC + cid
    b = wid // WPB
    s0 = lax.rem(wid, WPB) * ROWS_PER_W
    spbase = sid * ROWS_PER_W

    idx_cp = pltpu.async_copy(x_hbm.at[b, pl.ds(s0, ROWS_PER_W)], idx_v, sem_in)
    pos_cps = [
        pltpu.async_copy(
            pos_hbm.at[pl.ds(s0 + c * CH, CH)],
            acc_sh.at[pl.ds(spbase + c * CH, CH)],
            sems[c],
        )
        for c in range(NCHUNK)
    ]

    lane = lax.iota(jnp.int32, 16)
    for j in range(NCHUNK):
        for k in range(CH // LANES):
            ids_v[j, pl.ds(k * LANES, LANES)] = lane + (
                spbase + j * CH + k * LANES
            )

    idx_cp.wait()
    g_cps = [
        pltpu.async_copy(
            tok_hbm.at[idx_v.at[pl.ds(c * CH, CH)]],
            tok_v.at[pl.ds(c * CH, CH)],
            sems[c],
        )
        for c in range(NCHUNK)
    ]

    sa_cps = []
    for c in range(NCHUNK):
        pos_cps[c].wait()
        g_cps[c].wait()
        sa_cps.append(
            pltpu.async_copy(
                tok_v.at[pl.ds(c * CH, CH)],
                acc_sh.at[ids_v.at[c]],
                sems[c],
                add=True,
            )
        )
        if c >= 1:
            sa_cps[c - 1].wait()
            pltpu.async_copy(
                acc_sh.at[pl.ds(spbase + (c - 1) * CH, CH)],
                out_hbm.at[b, pl.ds(s0 + (c - 1) * CH, CH)],
                sem_out,
            )
    sa_cps[NCHUNK - 1].wait()
    last_out = pltpu.async_copy(
        acc_sh.at[pl.ds(spbase + (NCHUNK - 1) * CH, CH)],
        out_hbm.at[b, pl.ds(s0 + (NCHUNK - 1) * CH, CH)],
        sem_out,
    )
    for _ in range(NCHUNK):
        last_out.wait()


def kernel(x, tok_table, pos_table):
    return _embed_kernel(x, tok_table, pos_table)
